# baseline (speedup 1.0000x reference)
import functools

import jax
import jax.numpy as jnp
from jax import lax
from jax.experimental import pallas as pl
from jax.experimental.pallas import tpu as pltpu
from jax.experimental.pallas import tpu_sc as plsc

N = 10000
NP = 10240
E = 320000
D_IN = 128
HID = 64
NEG = 0.2

NC = 2
NS = 16
NW = NC * NS
EPT = E // NW
CK = 128
EPTS = 10240
CH = EPTS // CK
DUMMY = N + 100
DCAP = 2048
DCH = DCAP // CK
NB = 4

ROWS_BLK = 1000
N_BLOCKS = N // ROWS_BLK

_SC_PARAMS = pltpu.CompilerParams(
    needs_layout_passes=False, use_tc_tiling_on_sc=False)


def _type_of_block(i):
    return ((i >= 4).astype(jnp.int32) + (i >= 7).astype(jnp.int32)
            + (i >= 9).astype(jnp.int32))


def _dense0_body(x_ref, w_ref, b_ref, wg_ref, al_ref, ar_ref,
                 fl_ref, el_ref, er_ref):
    x = x_ref[...]
    h = jnp.dot(x, w_ref[0], preferred_element_type=jnp.float32) + b_ref[0]
    fl = jnp.dot(h, wg_ref[...], preferred_element_type=jnp.float32)
    fl_ref[...] = fl
    el_ref[...] = jnp.sum(fl * al_ref[...], axis=1, keepdims=True)
    er_ref[...] = jnp.sum(fl * ar_ref[...], axis=1, keepdims=True)


def _dense0(x, wstack, bstack, wg, al, ar):
    return pl.pallas_call(
        _dense0_body,
        grid=(N_BLOCKS,),
        in_specs=[
            pl.BlockSpec((ROWS_BLK, D_IN), lambda i: (i, 0)),
            pl.BlockSpec((1, D_IN, HID), lambda i: (_type_of_block(i), 0, 0)),
            pl.BlockSpec((1, 1, HID), lambda i: (_type_of_block(i), 0, 0)),
            pl.BlockSpec((HID, HID), lambda i: (0, 0)),
            pl.BlockSpec((1, HID), lambda i: (0, 0)),
            pl.BlockSpec((1, HID), lambda i: (0, 0)),
        ],
        out_specs=[
            pl.BlockSpec((ROWS_BLK, HID), lambda i: (i, 0)),
            pl.BlockSpec((ROWS_BLK, 1), lambda i: (i, 0)),
            pl.BlockSpec((ROWS_BLK, 1), lambda i: (i, 0)),
        ],
        out_shape=[
            jax.ShapeDtypeStruct((N, HID), jnp.float32),
            jax.ShapeDtypeStruct((N, 1), jnp.float32),
            jax.ShapeDtypeStruct((N, 1), jnp.float32),
        ],
    )(x, wstack, bstack, wg, al, ar)


def _mid_body(s_ref, es_ref, wg_ref, al_ref, ar_ref, fl_ref, el_ref, er_ref):
    ssum = s_ref[0] + s_ref[1]
    es = es_ref[0] + es_ref[1]
    rst = ssum / jnp.maximum(es, 1e-9)
    h1 = jnp.where(rst > 0, rst, jnp.exp(rst) - 1.0)
    fl = jnp.dot(h1, wg_ref[...], preferred_element_type=jnp.float32)
    fl_ref[...] = fl
    el_ref[...] = jnp.sum(fl * al_ref[...], axis=1, keepdims=True)
    er_ref[...] = jnp.sum(fl * ar_ref[...], axis=1, keepdims=True)


def _mid(s, es, wg, al, ar):
    return pl.pallas_call(
        _mid_body,
        grid=(N_BLOCKS,),
        in_specs=[
            pl.BlockSpec((NC, ROWS_BLK, HID), lambda i: (0, i, 0)),
            pl.BlockSpec((NC, ROWS_BLK, 1), lambda i: (0, i, 0)),
            pl.BlockSpec((HID, HID), lambda i: (0, 0)),
            pl.BlockSpec((1, HID), lambda i: (0, 0)),
            pl.BlockSpec((1, HID), lambda i: (0, 0)),
        ],
        out_specs=[
            pl.BlockSpec((ROWS_BLK, HID), lambda i: (i, 0)),
            pl.BlockSpec((ROWS_BLK, 1), lambda i: (i, 0)),
            pl.BlockSpec((ROWS_BLK, 1), lambda i: (i, 0)),
        ],
        out_shape=[
            jax.ShapeDtypeStruct((N, HID), jnp.float32),
            jax.ShapeDtypeStruct((N, 1), jnp.float32),
            jax.ShapeDtypeStruct((N, 1), jnp.float32),
        ],
    )(s, es, wg, al, ar)


def _norm_body(s_ref, es_ref, out_ref):
    es = es_ref[0] + es_ref[1]
    out_ref[...] = (s_ref[0] + s_ref[1]) / jnp.maximum(es, 1e-9)


def _norm(s, es):
    return pl.pallas_call(
        _norm_body,
        grid=(N_BLOCKS,),
        in_specs=[
            pl.BlockSpec((NC, ROWS_BLK, HID), lambda i: (0, i, 0)),
            pl.BlockSpec((NC, ROWS_BLK, 1), lambda i: (0, i, 0)),
        ],
        out_specs=pl.BlockSpec((ROWS_BLK, HID), lambda i: (i, 0)),
        out_shape=jax.ShapeDtypeStruct((N, HID), jnp.float32),
    )(s, es)


def _comb_body(s_ref, out_ref):
    out_ref[...] = s_ref[0] + s_ref[1]


def _comb(s):
    return pl.pallas_call(
        _comb_body,
        grid=(N_BLOCKS,),
        in_specs=[pl.BlockSpec((NC, ROWS_BLK, HID), lambda i: (0, i, 0))],
        out_specs=pl.BlockSpec((ROWS_BLK, HID), lambda i: (i, 0)),
        out_shape=jax.ShapeDtypeStruct((N, HID), jnp.float32),
    )(s)


def _sc_mesh():
    return plsc.VectorSubcoreMesh(
        core_axis_name="c", subcore_axis_name="s",
        num_cores=NC, num_subcores=NS)


def _splat16(v):
    return jnp.broadcast_to(v, (16,)).astype(jnp.int32)


def _edge_scalar_body(with_w, *refs):
    if with_w:
        (el_h, er_h, src_h, dst_h, zn_h, ef_h,
         ee_o, es_o, wc_o, srcd_o, dstd_o,
         acc, el_v, er_v, src2, dst2, ee2, ef2, wc2, srcd_v, dstd_v) = refs
    else:
        (el_h, er_h, src_h, dst_h, zn_h,
         ee_o, es_o,
         acc, el_v, er_v, src2, dst2, ee2) = refs
    c = lax.axis_index("c")
    s = lax.axis_index("s")
    npt = NP // NS
    pltpu.sync_copy(zn_h.at[pl.ds(s * npt, npt)], acc.at[pl.ds(s * npt, npt)])
    pltpu.sync_copy(el_h, el_v)
    pltpu.sync_copy(er_h, er_v)
    wid = c * NS + s
    pltpu.sync_copy(src_h.at[wid], src2)
    pltpu.sync_copy(dst_h.at[wid], dst2)
    if with_w:
        pltpu.sync_copy(ef_h.at[wid], ef2)

        @pl.loop(0, DCAP // 16)
        def _(i):
            srcd_v[pl.ds(i * 16, 16)] = jnp.zeros((16,), jnp.int32)
            dstd_v[pl.ds(i * 16, 16)] = jnp.full((16,), DUMMY, jnp.int32)
    plsc.subcore_barrier()

    def group(j, g, cnt):
        sl = pl.ds(g * 16, 16)
        sv = src2[j, sl]
        dv = dst2[j, sl]
        x = plsc.load_gather(el_v, [sv]) + plsc.load_gather(er_v, [dv])
        x = jnp.where(x > 0, x, x * NEG)
        ee2[j, sl] = jnp.exp(x)
        if with_w:
            ef = ef2[j, sl]
            wc2[j, sl] = (1.0
                          + jnp.where(ef == 0, 1.0, 0.0)
                          + jnp.where(ef == 4, 1.0, 0.0)
                          + jnp.where(ef == 5, 1.0, 0.0))
            msk = ef == 3
            plsc.store_compressed(srcd_v.at[pl.ds(cnt, 16)], sv, mask=msk)
            plsc.store_compressed(dstd_v.at[pl.ds(cnt, 16)], dv, mask=msk)
            cnt = cnt + jnp.sum(msk.astype(jnp.int32))
        return cnt

    def scat(j):
        pltpu.sync_copy(ee2.at[j], acc.at[dst2.at[j]], add=True)

    @pl.loop(0, CH - 2, init_carry=jnp.int32(0))
    def cnt(j, cnt):
        for g in range(8):
            cnt = group(j, g, cnt)
        scat(j)
        return cnt

    cnt = group(CH - 2, 0, cnt)
    zero16 = jnp.zeros((16,), jnp.float32)
    for g in range(1, 8):
        ee2[CH - 2, pl.ds(g * 16, 16)] = zero16
        if with_w:
            wc2[CH - 2, pl.ds(g * 16, 16)] = zero16
    scat(CH - 2)
    for g in range(8):
        ee2[CH - 1, pl.ds(g * 16, 16)] = zero16
        if with_w:
            wc2[CH - 1, pl.ds(g * 16, 16)] = zero16

    pltpu.sync_copy(ee2, ee_o.at[wid])
    if with_w:
        pltpu.sync_copy(wc2, wc_o.at[wid])
        pltpu.sync_copy(srcd_v, srcd_o.at[pl.ds(wid * DCAP, DCAP)])
        pltpu.sync_copy(dstd_v, dstd_o.at[pl.ds(wid * DCAP, DCAP)])
    plsc.subcore_barrier()
    pltpu.sync_copy(acc.at[pl.ds(s * npt, npt)],
                    es_o.at[c, pl.ds(s * npt, npt)])


def _edge_scalar(with_w):
    outs = [jax.ShapeDtypeStruct((NW, CH, CK), jnp.float32),
            jax.ShapeDtypeStruct((NC, NP), jnp.float32)]
    scr = [pltpu.VMEM_SHARED((NP,), jnp.float32),
           pltpu.VMEM((NP,), jnp.float32),
           pltpu.VMEM((NP,), jnp.float32),
           pltpu.VMEM((CH, CK), jnp.int32),
           pltpu.VMEM((CH, CK), jnp.int32),
           pltpu.VMEM((CH, CK), jnp.float32)]
    if with_w:
        outs += [jax.ShapeDtypeStruct((NW, CH, CK), jnp.float32),
                 jax.ShapeDtypeStruct((NW * DCAP,), jnp.int32),
                 jax.ShapeDtypeStruct((NW * DCAP,), jnp.int32)]
        scr += [pltpu.VMEM((CH, CK), jnp.int32),
                pltpu.VMEM((CH, CK), jnp.float32),
                pltpu.VMEM((DCAP,), jnp.int32),
                pltpu.VMEM((DCAP,), jnp.int32)]
    return pl.kernel(functools.partial(_edge_scalar_body, with_w),
                     out_type=outs, mesh=_sc_mesh(), scratch_types=scr,
                     compiler_params=_SC_PARAMS)


def _row_pass_body(weighted, nch, *refs):
    if weighted:
        (table_h, src_h, dst_h, w_h, z_h, out_h,
         acc, src2, dst2, w2) = refs[:10]
        rest = refs[10:]
    else:
        (table_h, src_h, dst_h, z_h, out_h,
         acc, src2, dst2) = refs[:8]
        w2 = None
        rest = refs[8:]
    bufs = rest[:NB]
    gsems = rest[NB:2 * NB]
    ssems = rest[2 * NB:3 * NB]
    c = lax.axis_index("c")
    s = lax.axis_index("s")
    npt = NP // NS
    wid = c * NS + s
    pltpu.sync_copy(src_h.at[wid], src2)
    pltpu.sync_copy(dst_h.at[wid], dst2)
    if weighted:
        pltpu.sync_copy(w_h.at[wid], w2)

    zero16 = jnp.zeros((16,), jnp.float32)

    @pl.loop(0, CK)
    def _(k):
        for q in range(4):
            bufs[0][k, pl.ds(q * 16, 16)] = zero16

    for t in range(npt // CK):
        pltpu.sync_copy(bufs[0], acc.at[pl.ds(s * npt + t * CK, CK)])
    plsc.subcore_barrier()

    @pl.loop(0, nch // NB)
    def _(j0):
        j = j0 * NB
        gds = [pltpu.async_copy(table_h.at[src2.at[j + b]], bufs[b], gsems[b])
               for b in range(NB)]
        sds = []
        for b in range(NB):
            gds[b].wait()
            if weighted:
                pass
            sds.append(pltpu.async_copy(
                bufs[b], acc.at[dst2.at[j + b]], ssems[b], add=True))
        for d in sds:
            d.wait()

    plsc.subcore_barrier()
    wds = []
    for t in range(npt // CK):
        b = t % 2
        if t >= 2:
            wds[t - 2].wait()
        pltpu.sync_copy(acc.at[pl.ds(s * npt + t * CK, CK)], bufs[b])
        wds.append(pltpu.async_copy(
            bufs[b], out_h.at[c, pl.ds(s * npt + t * CK, CK)], gsems[b]))
    for d in wds[-2:]:
        d.wait()


def _row_pass(weighted, nch):
    scr = [pltpu.VMEM_SHARED((NP, HID), jnp.float32),
           pltpu.VMEM((nch, CK), jnp.int32),
           pltpu.VMEM((nch, CK), jnp.int32)]
    if weighted:
        scr += [pltpu.VMEM((nch, CK), jnp.float32)]
    scr += [pltpu.VMEM((CK, HID), jnp.float32)] * NB
    scr += [pltpu.SemaphoreType.DMA] * (2 * NB)
    return pl.kernel(
        functools.partial(_row_pass_body, weighted, nch),
        out_type=jax.ShapeDtypeStruct((NC, NP, HID), jnp.float32),
        mesh=_sc_mesh(), scratch_types=scr, compiler_params=_SC_PARAMS)


def _pad_np(v):
    return jnp.concatenate([v, jnp.zeros((NP - N,), jnp.float32)])


def _pad_edges(v, fill):
    return jnp.pad(v.reshape(NW, EPT), ((0, 0), (0, EPTS - EPT)),
                   constant_values=fill).reshape(NW, CH, CK)


def kernel(feat0, feat1, feat2, feat3, edge_index, e_feat,
           W_fc0, b_fc0, W_fc1, b_fc1, W_fc2, b_fc2, W_fc3, b_fc3,
           W_g0, attn_l0, attn_r0, W_g1, attn_l1, attn_r1):
    srcp = _pad_edges(edge_index[0], 0)
    dstp = _pad_edges(edge_index[1], DUMMY)
    efp = _pad_edges(e_feat, 6)
    x = jnp.concatenate([feat0, feat1, feat2, feat3], axis=0)
    wstack = jnp.stack([W_fc0, W_fc1, W_fc2, W_fc3])
    bstack = jnp.stack([b_fc0, b_fc1, b_fc2, b_fc3]).reshape(4, 1, HID)
    zn = jnp.zeros((NP,), jnp.float32)
    z64 = jnp.zeros((NP, HID), jnp.float32)

    fl0, el0, er0 = _dense0(x, wstack, bstack, W_g0, attn_l0, attn_r0)

    ee0, es0, wc, srcd, dstd = _edge_scalar(True)(
        _pad_np(el0[:, 0]), _pad_np(er0[:, 0]), srcp, dstp, zn, efp)
    s0 = _row_pass(True, CH)(fl0, srcp, dstp, ee0, z64)

    fl1, el1, er1 = _mid(s0, es0.reshape(NC, NP, 1), W_g1, attn_l1, attn_r1)

    ee1, es1 = _edge_scalar(False)(
        _pad_np(el1[:, 0]), _pad_np(er1[:, 0]), srcp, dstp, zn)
    s1 = _row_pass(True, CH)(fl1, srcp, dstp, ee1, z64)

    h2 = _norm(s1, es1.reshape(NC, NP, 1))

    ftp = _row_pass(True, CH)(h2, srcp, dstp, wc, z64)
    ft = _comb(ftp)

    srcd3 = srcd.reshape(NW, DCH, CK)
    dstd3 = dstd.reshape(NW, DCH, CK)
    outp = _row_pass(False, DCH)(ft, srcd3, dstd3, z64)
    return _comb(outp)


# speedup vs baseline: 1.5943x; 1.5943x over previous
import functools

import jax
import jax.numpy as jnp
from jax import lax
from jax.experimental import pallas as pl
from jax.experimental.pallas import tpu as pltpu
from jax.experimental.pallas import tpu_sc as plsc

N = 10000
NP = 10240
E = 320000
D_IN = 128
HID = 64
NEG = 0.2

NC = 2
NS = 16
NW = NC * NS
EPT = E // NW
CK = 128
EPTS = 10240
CH = EPTS // CK
DUMMY = N + 100
DCAP = 2048
DCH = DCAP // CK
NB = 2

ROWS_BLK = 1000
N_BLOCKS = N // ROWS_BLK

_SC_PARAMS = pltpu.CompilerParams(
    needs_layout_passes=False, use_tc_tiling_on_sc=False)


def _type_of_block(i):
    return ((i >= 4).astype(jnp.int32) + (i >= 7).astype(jnp.int32)
            + (i >= 9).astype(jnp.int32))


def _dense0_body(x_ref, w_ref, b_ref, wg_ref, al_ref, ar_ref,
                 fl_ref, el_ref, er_ref):
    x = x_ref[...]
    h = jnp.dot(x, w_ref[0], preferred_element_type=jnp.float32) + b_ref[0]
    fl = jnp.dot(h, wg_ref[...], preferred_element_type=jnp.float32)
    fl_ref[...] = fl
    el_ref[...] = jnp.sum(fl * al_ref[...], axis=1, keepdims=True)
    er_ref[...] = jnp.sum(fl * ar_ref[...], axis=1, keepdims=True)


def _dense0(x, wstack, bstack, wg, al, ar):
    return pl.pallas_call(
        _dense0_body,
        grid=(N_BLOCKS,),
        in_specs=[
            pl.BlockSpec((ROWS_BLK, D_IN), lambda i: (i, 0)),
            pl.BlockSpec((1, D_IN, HID), lambda i: (_type_of_block(i), 0, 0)),
            pl.BlockSpec((1, 1, HID), lambda i: (_type_of_block(i), 0, 0)),
            pl.BlockSpec((HID, HID), lambda i: (0, 0)),
            pl.BlockSpec((1, HID), lambda i: (0, 0)),
            pl.BlockSpec((1, HID), lambda i: (0, 0)),
        ],
        out_specs=[
            pl.BlockSpec((ROWS_BLK, HID), lambda i: (i, 0)),
            pl.BlockSpec((ROWS_BLK, 1), lambda i: (i, 0)),
            pl.BlockSpec((ROWS_BLK, 1), lambda i: (i, 0)),
        ],
        out_shape=[
            jax.ShapeDtypeStruct((NP, HID), jnp.float32),
            jax.ShapeDtypeStruct((N, 1), jnp.float32),
            jax.ShapeDtypeStruct((N, 1), jnp.float32),
        ],
    )(x, wstack, bstack, wg, al, ar)


def _mid_body(s_ref, es_ref, wg_ref, al_ref, ar_ref, fl_ref, el_ref, er_ref):
    ssum = s_ref[0] + s_ref[1]
    es = es_ref[0] + es_ref[1]
    rst = ssum / jnp.maximum(es, 1e-9)
    h1 = jnp.where(rst > 0, rst, jnp.exp(rst) - 1.0)
    fl = jnp.dot(h1, wg_ref[...], preferred_element_type=jnp.float32)
    fl_ref[...] = fl
    el_ref[...] = jnp.sum(fl * al_ref[...], axis=1, keepdims=True)
    er_ref[...] = jnp.sum(fl * ar_ref[...], axis=1, keepdims=True)


def _mid(s, es, wg, al, ar):
    return pl.pallas_call(
        _mid_body,
        grid=(N_BLOCKS,),
        in_specs=[
            pl.BlockSpec((NC, ROWS_BLK, HID), lambda i: (0, i, 0)),
            pl.BlockSpec((NC, ROWS_BLK, 1), lambda i: (0, i, 0)),
            pl.BlockSpec((HID, HID), lambda i: (0, 0)),
            pl.BlockSpec((1, HID), lambda i: (0, 0)),
            pl.BlockSpec((1, HID), lambda i: (0, 0)),
        ],
        out_specs=[
            pl.BlockSpec((ROWS_BLK, HID), lambda i: (i, 0)),
            pl.BlockSpec((ROWS_BLK, 1), lambda i: (i, 0)),
            pl.BlockSpec((ROWS_BLK, 1), lambda i: (i, 0)),
        ],
        out_shape=[
            jax.ShapeDtypeStruct((NP, HID), jnp.float32),
            jax.ShapeDtypeStruct((N, 1), jnp.float32),
            jax.ShapeDtypeStruct((N, 1), jnp.float32),
        ],
    )(s, es, wg, al, ar)


def _norm_body(s_ref, es_ref, out_ref):
    es = es_ref[0] + es_ref[1]
    out_ref[...] = (s_ref[0] + s_ref[1]) / jnp.maximum(es, 1e-9)


def _norm(s, es):
    return pl.pallas_call(
        _norm_body,
        grid=(N_BLOCKS,),
        in_specs=[
            pl.BlockSpec((NC, ROWS_BLK, HID), lambda i: (0, i, 0)),
            pl.BlockSpec((NC, ROWS_BLK, 1), lambda i: (0, i, 0)),
        ],
        out_specs=pl.BlockSpec((ROWS_BLK, HID), lambda i: (i, 0)),
        out_shape=jax.ShapeDtypeStruct((NP, HID), jnp.float32),
    )(s, es)


def _comb_body(s_ref, out_ref):
    out_ref[...] = s_ref[0] + s_ref[1]


def _comb(s, out_n=N):
    return pl.pallas_call(
        _comb_body,
        grid=(N_BLOCKS,),
        in_specs=[pl.BlockSpec((NC, ROWS_BLK, HID), lambda i: (0, i, 0))],
        out_specs=pl.BlockSpec((ROWS_BLK, HID), lambda i: (i, 0)),
        out_shape=jax.ShapeDtypeStruct((out_n, HID), jnp.float32),
    )(s)


def _sc_mesh():
    return plsc.VectorSubcoreMesh(
        core_axis_name="c", subcore_axis_name="s",
        num_cores=NC, num_subcores=NS)


def _splat16(v):
    return jnp.broadcast_to(v, (16,)).astype(jnp.int32)


def _edge_scalar_body(with_w, *refs):
    if with_w:
        (el_h, er_h, src_h, dst_h, zn_h, ef_h,
         ee_o, es_o, wc_o, srcd_o, dstd_o,
         acc, el_v, er_v, src2, dst2, ee2, ef2, wc2, srcd_v, dstd_v) = refs
    else:
        (el_h, er_h, src_h, dst_h, zn_h,
         ee_o, es_o,
         acc, el_v, er_v, src2, dst2, ee2) = refs
    c = lax.axis_index("c")
    s = lax.axis_index("s")
    npt = NP // NS
    pltpu.sync_copy(zn_h.at[pl.ds(s * npt, npt)], acc.at[pl.ds(s * npt, npt)])
    pltpu.sync_copy(el_h, el_v)
    pltpu.sync_copy(er_h, er_v)
    wid = c * NS + s
    pltpu.sync_copy(src_h.at[wid], src2)
    pltpu.sync_copy(dst_h.at[wid], dst2)
    if with_w:
        pltpu.sync_copy(ef_h.at[wid], ef2)

        @pl.loop(0, DCAP // 16)
        def _(i):
            srcd_v[pl.ds(i * 16, 16)] = jnp.zeros((16,), jnp.int32)
            dstd_v[pl.ds(i * 16, 16)] = jnp.full((16,), DUMMY, jnp.int32)
    plsc.subcore_barrier()

    def group(j, g, cnt):
        sl = pl.ds(g * 16, 16)
        sv = src2[j, sl]
        dv = dst2[j, sl]
        x = plsc.load_gather(el_v, [sv]) + plsc.load_gather(er_v, [dv])
        x = jnp.where(x > 0, x, x * NEG)
        ee2[j, sl] = jnp.exp(x)
        if with_w:
            ef = ef2[j, sl]
            wc2[j, sl] = (1.0
                          + jnp.where(ef == 0, 1.0, 0.0)
                          + jnp.where(ef == 4, 1.0, 0.0)
                          + jnp.where(ef == 5, 1.0, 0.0))
            msk = ef == 3
            plsc.store_compressed(srcd_v.at[pl.ds(cnt, 16)], sv, mask=msk)
            plsc.store_compressed(dstd_v.at[pl.ds(cnt, 16)], dv, mask=msk)
            cnt = cnt + jnp.sum(msk.astype(jnp.int32))
        return cnt

    def scat(j):
        pltpu.sync_copy(ee2.at[j], acc.at[dst2.at[j]], add=True)

    @pl.loop(0, CH - 2, init_carry=jnp.int32(0))
    def cnt(j, cnt):
        for g in range(8):
            cnt = group(j, g, cnt)
        scat(j)
        return cnt

    cnt = group(CH - 2, 0, cnt)
    zero16 = jnp.zeros((16,), jnp.float32)
    for g in range(1, 8):
        ee2[CH - 2, pl.ds(g * 16, 16)] = zero16
        if with_w:
            wc2[CH - 2, pl.ds(g * 16, 16)] = zero16
    scat(CH - 2)
    for g in range(8):
        ee2[CH - 1, pl.ds(g * 16, 16)] = zero16
        if with_w:
            wc2[CH - 1, pl.ds(g * 16, 16)] = zero16

    pltpu.sync_copy(ee2, ee_o.at[wid])
    if with_w:
        pltpu.sync_copy(wc2, wc_o.at[wid])
        pltpu.sync_copy(srcd_v, srcd_o.at[pl.ds(wid * DCAP, DCAP)])
        pltpu.sync_copy(dstd_v, dstd_o.at[pl.ds(wid * DCAP, DCAP)])
    plsc.subcore_barrier()
    pltpu.sync_copy(acc.at[pl.ds(s * npt, npt)],
                    es_o.at[c, pl.ds(s * npt, npt)])


def _edge_scalar(with_w):
    outs = [jax.ShapeDtypeStruct((NW, CH, CK), jnp.float32),
            jax.ShapeDtypeStruct((NC, NP), jnp.float32)]
    scr = [pltpu.VMEM_SHARED((NP,), jnp.float32),
           pltpu.VMEM((NP,), jnp.float32),
           pltpu.VMEM((NP,), jnp.float32),
           pltpu.VMEM((CH, CK), jnp.int32),
           pltpu.VMEM((CH, CK), jnp.int32),
           pltpu.VMEM((CH, CK), jnp.float32)]
    if with_w:
        outs += [jax.ShapeDtypeStruct((NW, CH, CK), jnp.float32),
                 jax.ShapeDtypeStruct((NW * DCAP,), jnp.int32),
                 jax.ShapeDtypeStruct((NW * DCAP,), jnp.int32)]
        scr += [pltpu.VMEM((CH, CK), jnp.int32),
                pltpu.VMEM((CH, CK), jnp.float32),
                pltpu.VMEM((DCAP,), jnp.int32),
                pltpu.VMEM((DCAP,), jnp.int32)]
    return pl.kernel(functools.partial(_edge_scalar_body, with_w),
                     out_type=outs, mesh=_sc_mesh(), scratch_types=scr,
                     compiler_params=_SC_PARAMS)


def _row_pass_body(weighted, nch, *refs):
    if weighted:
        (table_h, src_h, dst_h, w_h, z_h, out_h,
         acc, tbl, src2, dst2, w2) = refs[:11]
        rest = refs[11:]
    else:
        (table_h, src_h, dst_h, z_h, out_h,
         acc, tbl, src2, dst2) = refs[:9]
        w2 = None
        rest = refs[9:]
    bufs = rest[:NB]
    gsems = rest[NB:2 * NB]
    ssems = rest[2 * NB:3 * NB]
    c = lax.axis_index("c")
    s = lax.axis_index("s")
    npt = NP // NS
    wid = c * NS + s
    pltpu.sync_copy(src_h.at[wid], src2)
    pltpu.sync_copy(dst_h.at[wid], dst2)
    if weighted:
        pltpu.sync_copy(w_h.at[wid], w2)

    zero16 = jnp.zeros((16,), jnp.float32)

    @pl.loop(0, CK)
    def _(k):
        for q in range(4):
            bufs[0][k, pl.ds(q * 16, 16)] = zero16

    for t in range(npt // CK):
        pltpu.sync_copy(bufs[0], acc.at[pl.ds(s * npt + t * CK, CK)])
    tds = []
    for t in range(npt // CK):
        b = t % 2
        if t >= 2:
            tds[t - 2].wait()
        pltpu.sync_copy(table_h.at[pl.ds(s * npt + t * CK, CK)], bufs[b])
        tds.append(pltpu.async_copy(
            bufs[b], tbl.at[pl.ds(s * npt + t * CK, CK)], gsems[b]))
    for d in tds[-2:]:
        d.wait()
    plsc.subcore_barrier()

    @pl.loop(0, nch // NB)
    def _(j0):
        j = j0 * NB
        gds = [pltpu.async_copy(tbl.at[src2.at[j + b]], bufs[b], gsems[b])
               for b in range(NB)]
        sds = []
        for b in range(NB):
            gds[b].wait()
            if weighted:
                jc = j + b
                buf = bufs[b]

                @pl.loop(0, CK, unroll=4)
                def _(k, _jc=jc, _buf=buf):
                    wk = plsc.load_gather(w2, [_splat16(_jc), _splat16(k)])
                    for q in range(4):
                        sl = pl.ds(q * 16, 16)
                        _buf[k, sl] = _buf[k, sl] * wk
            sds.append(pltpu.async_copy(
                bufs[b], acc.at[dst2.at[j + b]], ssems[b], add=True))
        for d in sds:
            d.wait()

    plsc.subcore_barrier()
    wds = []
    for t in range(npt // CK):
        b = t % 2
        if t >= 2:
            wds[t - 2].wait()
        pltpu.sync_copy(acc.at[pl.ds(s * npt + t * CK, CK)], bufs[b])
        wds.append(pltpu.async_copy(
            bufs[b], out_h.at[c, pl.ds(s * npt + t * CK, CK)], gsems[b]))
    for d in wds[-2:]:
        d.wait()


def _row_pass(weighted, nch):
    scr = [pltpu.VMEM_SHARED((NP, HID), jnp.float32),
           pltpu.VMEM_SHARED((NP, HID), jnp.float32),
           pltpu.VMEM((nch, CK), jnp.int32),
           pltpu.VMEM((nch, CK), jnp.int32)]
    if weighted:
        scr += [pltpu.VMEM((nch, CK), jnp.float32)]
    scr += [pltpu.VMEM((CK, HID), jnp.float32)] * NB
    scr += [pltpu.SemaphoreType.DMA] * (2 * NB)
    return pl.kernel(
        functools.partial(_row_pass_body, weighted, nch),
        out_type=jax.ShapeDtypeStruct((NC, NP, HID), jnp.float32),
        mesh=_sc_mesh(), scratch_types=scr, compiler_params=_SC_PARAMS)


def _pad_np(v):
    return jnp.concatenate([v, jnp.zeros((NP - N,), jnp.float32)])


def _pad_edges(v, fill):
    return jnp.pad(v.reshape(NW, EPT), ((0, 0), (0, EPTS - EPT)),
                   constant_values=fill).reshape(NW, CH, CK)


def kernel(feat0, feat1, feat2, feat3, edge_index, e_feat,
           W_fc0, b_fc0, W_fc1, b_fc1, W_fc2, b_fc2, W_fc3, b_fc3,
           W_g0, attn_l0, attn_r0, W_g1, attn_l1, attn_r1):
    srcp = _pad_edges(edge_index[0], 0)
    dstp = _pad_edges(edge_index[1], DUMMY)
    efp = _pad_edges(e_feat, 6)
    x = jnp.concatenate([feat0, feat1, feat2, feat3], axis=0)
    wstack = jnp.stack([W_fc0, W_fc1, W_fc2, W_fc3])
    bstack = jnp.stack([b_fc0, b_fc1, b_fc2, b_fc3]).reshape(4, 1, HID)
    zn = jnp.zeros((NP,), jnp.float32)
    z64 = jnp.zeros((NP, HID), jnp.float32)

    fl0, el0, er0 = _dense0(x, wstack, bstack, W_g0, attn_l0, attn_r0)

    ee0, es0, wc, srcd, dstd = _edge_scalar(True)(
        _pad_np(el0[:, 0]), _pad_np(er0[:, 0]), srcp, dstp, zn, efp)
    s0 = _row_pass(True, CH)(fl0, srcp, dstp, ee0, z64)

    fl1, el1, er1 = _mid(s0, es0.reshape(NC, NP, 1), W_g1, attn_l1, attn_r1)

    ee1, es1 = _edge_scalar(False)(
        _pad_np(el1[:, 0]), _pad_np(er1[:, 0]), srcp, dstp, zn)
    s1 = _row_pass(True, CH)(fl1, srcp, dstp, ee1, z64)

    h2 = _norm(s1, es1.reshape(NC, NP, 1))

    ftp = _row_pass(True, CH)(h2, srcp, dstp, wc, z64)
    ft = _comb(ftp, NP)

    srcd3 = srcd.reshape(NW, DCH, CK)
    dstd3 = dstd.reshape(NW, DCH, CK)
    outp = _row_pass(False, DCH)(ft, srcd3, dstd3, z64)
    return _comb(outp)
